# baseline (speedup 1.0000x reference)
import functools

import jax
import jax.numpy as jnp
from jax import lax
from jax.experimental import pallas as pl
from jax.experimental.pallas import tpu as pltpu
from jax.experimental.pallas import tpu_sc as plsc

B, T, H = 4, 2048, 1024
G, V, D = 2, 320, 128
N = B * T

TN1 = 2048
NT1 = N // TN1

NC, NS, L = 2, 16, 16
NW = NC * NS
CH = 1
NCHUNK = N // CH
TOK_W = NCHUNK // NW

TN2 = 1024
NT2 = NCHUNK // TN2


def _proj_argmax_kernel(x_ref, wpt_ref, bp_ref, i0_ref, i1_ref, perp_ref,
                        counts_ref):
    i = pl.program_id(0)

    @pl.when(i == 0)
    def _init():
        counts_ref[...] = jnp.zeros_like(counts_ref)

    logits = jnp.dot(x_ref[...], wpt_ref[...], preferred_element_type=jnp.float32)
    logits = logits + bp_ref[...]
    iota = lax.broadcasted_iota(jnp.int32, (TN1, V), 1)

    def first_idx(l):
        m = jnp.max(l, axis=1, keepdims=True)
        return jnp.min(jnp.where(l == m, iota, V), axis=1, keepdims=True)

    f0 = first_idx(logits[:, :V])
    f1 = first_idx(logits[:, V:])
    i0_ref[...] = f0.reshape(TN1 // 128, 128)
    i1_ref[...] = f1.reshape(TN1 // 128, 128)
    oh0 = (iota == f0).astype(jnp.float32)
    oh1 = (iota == f1).astype(jnp.float32)
    counts_ref[0:1, :] = counts_ref[0:1, :] + jnp.sum(oh0, axis=0, keepdims=True)
    counts_ref[1:2, :] = counts_ref[1:2, :] + jnp.sum(oh1, axis=0, keepdims=True)

    @pl.when(i == NT1 - 1)
    def _fin():
        avg = counts_ref[...] / N
        ent = -jnp.sum(avg * jnp.log(avg + 1e-9), axis=1, keepdims=True)
        perp_ref[...] = jnp.sum(jnp.exp(ent), axis=0, keepdims=True) / G


def _sc_gather_kernel(i0_hbm, i1_hbm, cb0_hbm, cb1_hbm,
                      g0_hbm, g1_hbm,
                      i0_v, i1_v, r0_v, r1_v, sem):
    c = lax.axis_index("c")
    s = lax.axis_index("s")
    w = s * NC + c
    base = w * TOK_W

    pltpu.sync_copy(i0_hbm.at[pl.ds(base, TOK_W)], i0_v)
    pltpu.sync_copy(i1_hbm.at[pl.ds(base, TOK_W)], i1_v)

    ci = min(128, TOK_W)
    copies = []
    for j in range(TOK_W // ci):
        sl = pl.ds(j * ci, ci)
        copies.append(pltpu.async_copy(cb0_hbm.at[i0_v.at[sl]], r0_v.at[sl], sem))
        copies.append(pltpu.async_copy(cb1_hbm.at[i1_v.at[sl]], r1_v.at[sl], sem))

    for cp in copies:
        cp.wait()

    pltpu.sync_copy(r0_v, g0_hbm.at[pl.ds(base, TOK_W)])
    pltpu.sync_copy(r1_v, g1_hbm.at[pl.ds(base, TOK_W)])


def _out_proj_kernel(g0_ref, g1_ref, x_ref, wot_ref, bo_ref,
                     q_ref, sse_out_ref, sse_ref):
    i = pl.program_id(0)

    @pl.when(i == 0)
    def _init():
        sse_ref[...] = jnp.zeros_like(sse_ref)

    wot_bf = wot_ref[...].astype(jnp.bfloat16)
    q = jnp.dot(g0_ref[...].astype(jnp.bfloat16), wot_bf[0:D, :],
                preferred_element_type=jnp.float32)
    q = q + jnp.dot(g1_ref[...].astype(jnp.bfloat16), wot_bf[D:, :],
                    preferred_element_type=jnp.float32)
    q = q + bo_ref[...]
    q_ref[...] = q
    r = (x_ref[...] - q) ** 2
    sse_ref[...] = sse_ref[...] + jnp.sum(r, axis=0, keepdims=True)

    @pl.when(i == NT2 - 1)
    def _fin():
        sse_out_ref[...] = jnp.sum(sse_ref[...], axis=1, keepdims=True) / (N * H)


@jax.jit
def kernel(x, Wp, bp, codebook, Wo, bo):
    x2d = x.reshape(N, H)
    wpt = Wp.T
    wot = Wo.T
    bp2d = bp.reshape(1, G * V)
    bo2d = bo.reshape(1, H)
    cb0 = codebook[0]
    cb1 = codebook[1]

    i0, i1, perp = pl.pallas_call(
        _proj_argmax_kernel,
        grid=(NT1,),
        in_specs=[
            pl.BlockSpec((TN1, H), lambda i: (i, 0)),
            pl.BlockSpec((H, G * V), lambda i: (0, 0)),
            pl.BlockSpec((1, G * V), lambda i: (0, 0)),
        ],
        out_specs=[
            pl.BlockSpec((TN1 // 128, 128), lambda i: (i, 0)),
            pl.BlockSpec((TN1 // 128, 128), lambda i: (i, 0)),
            pl.BlockSpec((1, 1), lambda i: (0, 0)),
        ],
        out_shape=[
            jax.ShapeDtypeStruct((N // 128, 128), jnp.int32),
            jax.ShapeDtypeStruct((N // 128, 128), jnp.int32),
            jax.ShapeDtypeStruct((1, 1), jnp.float32),
        ],
        scratch_shapes=[
            pltpu.VMEM((G, V), jnp.float32),
        ],
    )(x2d, wpt, bp2d)

    sc_gather = pl.kernel(
        _sc_gather_kernel,
        out_type=[
            jax.ShapeDtypeStruct((N, D), jnp.float32),
            jax.ShapeDtypeStruct((N, D), jnp.float32),
        ],
        mesh=plsc.VectorSubcoreMesh(core_axis_name="c", subcore_axis_name="s"),
        scratch_types=[
            pltpu.VMEM((TOK_W,), jnp.int32),
            pltpu.VMEM((TOK_W,), jnp.int32),
            pltpu.VMEM((TOK_W, D), jnp.float32),
            pltpu.VMEM((TOK_W, D), jnp.float32),
            pltpu.SemaphoreType.DMA,
        ],
    )
    i0f = i0.reshape(N)
    i1f = i1.reshape(N)

    out_proj = functools.partial(
        pl.pallas_call,
        _out_proj_kernel,
        grid=(NT2,),
        in_specs=[
            pl.BlockSpec((TN2, D), lambda i: (i, 0)),
            pl.BlockSpec((TN2, D), lambda i: (i, 0)),
            pl.BlockSpec((TN2, H), lambda i: (i, 0)),
            pl.BlockSpec((G * D, H), lambda i: (0, 0)),
            pl.BlockSpec((1, H), lambda i: (0, 0)),
        ],
        out_specs=[
            pl.BlockSpec((TN2, H), lambda i: (i, 0)),
            pl.BlockSpec((1, 1), lambda i: (0, 0)),
        ],
        out_shape=[
            jax.ShapeDtypeStruct((NCHUNK, H), jnp.float32),
            jax.ShapeDtypeStruct((1, 1), jnp.float32),
        ],
        scratch_shapes=[
            pltpu.VMEM((1, H), jnp.float32),
        ],
    )()

    g0, g1 = sc_gather(i0f, i1f, cb0, cb1)
    q2d, commit = out_proj(g0, g1, x2d, wot, bo2d)
    return q2d.reshape(B, T, H), perp[0, 0], commit[0, 0]


# speedup vs baseline: 1.0120x; 1.0120x over previous
import functools

import jax
import jax.numpy as jnp
from jax import lax
from jax.experimental import pallas as pl
from jax.experimental.pallas import tpu as pltpu
from jax.experimental.pallas import tpu_sc as plsc

B, T, H = 4, 2048, 1024
G, V, D = 2, 320, 128
N = B * T

TN1 = 2048
NT1 = N // TN1

NC, NS, L = 2, 16, 16
NW = NC * NS
CH = 1
NCHUNK = N // CH
TOK_W = NCHUNK // NW

TN2 = 2048
NT2 = NCHUNK // TN2


def _proj_argmax_kernel(x_ref, wpt_ref, bp_ref, i0_ref, i1_ref, perp_ref,
                        counts_ref):
    i = pl.program_id(0)

    @pl.when(i == 0)
    def _init():
        counts_ref[...] = jnp.zeros_like(counts_ref)

    logits = jnp.dot(x_ref[...], wpt_ref[...], preferred_element_type=jnp.float32)
    logits = logits + bp_ref[...]
    iota = lax.broadcasted_iota(jnp.int32, (TN1, V), 1)

    def first_idx(l):
        m = jnp.max(l, axis=1, keepdims=True)
        return jnp.min(jnp.where(l == m, iota, V), axis=1, keepdims=True)

    f0 = first_idx(logits[:, :V])
    f1 = first_idx(logits[:, V:])
    i0_ref[...] = f0.reshape(TN1 // 128, 128)
    i1_ref[...] = f1.reshape(TN1 // 128, 128)
    oh0 = (iota == f0).astype(jnp.float32)
    oh1 = (iota == f1).astype(jnp.float32)
    counts_ref[0:1, :] = counts_ref[0:1, :] + jnp.sum(oh0, axis=0, keepdims=True)
    counts_ref[1:2, :] = counts_ref[1:2, :] + jnp.sum(oh1, axis=0, keepdims=True)

    @pl.when(i == NT1 - 1)
    def _fin():
        avg = counts_ref[...] / N
        ent = -jnp.sum(avg * jnp.log(avg + 1e-9), axis=1, keepdims=True)
        perp_ref[...] = jnp.sum(jnp.exp(ent), axis=0, keepdims=True) / G


def _sc_gather_kernel(i0_hbm, i1_hbm, cb0_hbm, cb1_hbm,
                      g0_hbm, g1_hbm,
                      i0_v, i1_v, r0_v, r1_v, sem):
    c = lax.axis_index("c")
    s = lax.axis_index("s")
    w = s * NC + c
    base = w * TOK_W

    pltpu.sync_copy(i0_hbm.at[pl.ds(base, TOK_W)], i0_v)
    pltpu.sync_copy(i1_hbm.at[pl.ds(base, TOK_W)], i1_v)

    ci = min(128, TOK_W)
    copies = []
    for j in range(TOK_W // ci):
        sl = pl.ds(j * ci, ci)
        copies.append(pltpu.async_copy(cb0_hbm.at[i0_v.at[sl]], r0_v.at[sl], sem))
        copies.append(pltpu.async_copy(cb1_hbm.at[i1_v.at[sl]], r1_v.at[sl], sem))

    for cp in copies:
        cp.wait()

    pltpu.sync_copy(r0_v, g0_hbm.at[pl.ds(base, TOK_W)])
    pltpu.sync_copy(r1_v, g1_hbm.at[pl.ds(base, TOK_W)])


def _out_proj_kernel(g0_ref, g1_ref, x_ref, wot_ref, bo_ref,
                     q_ref, sse_out_ref, sse_ref):
    i = pl.program_id(0)

    @pl.when(i == 0)
    def _init():
        sse_ref[...] = jnp.zeros_like(sse_ref)

    wot_bf = wot_ref[...].astype(jnp.bfloat16)
    q = jnp.dot(g0_ref[...].astype(jnp.bfloat16), wot_bf[0:D, :],
                preferred_element_type=jnp.float32)
    q = q + jnp.dot(g1_ref[...].astype(jnp.bfloat16), wot_bf[D:, :],
                    preferred_element_type=jnp.float32)
    q = q + bo_ref[...]
    q_ref[...] = q
    r = (x_ref[...] - q) ** 2
    sse_ref[...] = sse_ref[...] + jnp.sum(r, axis=0, keepdims=True)

    @pl.when(i == NT2 - 1)
    def _fin():
        sse_out_ref[...] = jnp.sum(sse_ref[...], axis=1, keepdims=True) / (N * H)


@jax.jit
def kernel(x, Wp, bp, codebook, Wo, bo):
    x2d = x.reshape(N, H)
    wpt = Wp.T
    wot = Wo.T
    bp2d = bp.reshape(1, G * V)
    bo2d = bo.reshape(1, H)
    cb0 = codebook[0]
    cb1 = codebook[1]

    i0, i1, perp = pl.pallas_call(
        _proj_argmax_kernel,
        grid=(NT1,),
        in_specs=[
            pl.BlockSpec((TN1, H), lambda i: (i, 0)),
            pl.BlockSpec((H, G * V), lambda i: (0, 0)),
            pl.BlockSpec((1, G * V), lambda i: (0, 0)),
        ],
        out_specs=[
            pl.BlockSpec((TN1 // 128, 128), lambda i: (i, 0)),
            pl.BlockSpec((TN1 // 128, 128), lambda i: (i, 0)),
            pl.BlockSpec((1, 1), lambda i: (0, 0)),
        ],
        out_shape=[
            jax.ShapeDtypeStruct((N // 128, 128), jnp.int32),
            jax.ShapeDtypeStruct((N // 128, 128), jnp.int32),
            jax.ShapeDtypeStruct((1, 1), jnp.float32),
        ],
        scratch_shapes=[
            pltpu.VMEM((G, V), jnp.float32),
        ],
    )(x2d, wpt, bp2d)

    sc_gather = pl.kernel(
        _sc_gather_kernel,
        out_type=[
            jax.ShapeDtypeStruct((N, D), jnp.float32),
            jax.ShapeDtypeStruct((N, D), jnp.float32),
        ],
        mesh=plsc.VectorSubcoreMesh(core_axis_name="c", subcore_axis_name="s"),
        scratch_types=[
            pltpu.VMEM((TOK_W,), jnp.int32),
            pltpu.VMEM((TOK_W,), jnp.int32),
            pltpu.VMEM((TOK_W, D), jnp.float32),
            pltpu.VMEM((TOK_W, D), jnp.float32),
            pltpu.SemaphoreType.DMA,
        ],
    )
    i0f = i0.reshape(N)
    i1f = i1.reshape(N)

    out_proj = functools.partial(
        pl.pallas_call,
        _out_proj_kernel,
        grid=(NT2,),
        in_specs=[
            pl.BlockSpec((TN2, D), lambda i: (i, 0)),
            pl.BlockSpec((TN2, D), lambda i: (i, 0)),
            pl.BlockSpec((TN2, H), lambda i: (i, 0)),
            pl.BlockSpec((G * D, H), lambda i: (0, 0)),
            pl.BlockSpec((1, H), lambda i: (0, 0)),
        ],
        out_specs=[
            pl.BlockSpec((TN2, H), lambda i: (i, 0)),
            pl.BlockSpec((1, 1), lambda i: (0, 0)),
        ],
        out_shape=[
            jax.ShapeDtypeStruct((NCHUNK, H), jnp.float32),
            jax.ShapeDtypeStruct((1, 1), jnp.float32),
        ],
        scratch_shapes=[
            pltpu.VMEM((1, H), jnp.float32),
        ],
    )()

    g0, g1 = sc_gather(i0f, i1f, cb0, cb1)
    q2d, commit = out_proj(g0, g1, x2d, wot, bo2d)
    return q2d.reshape(B, T, H), perp[0, 0], commit[0, 0]
